# baseline (speedup 1.0000x reference)
import functools

import jax
import jax.numpy as jnp
from jax import lax
from jax.experimental import pallas as pl
from jax.experimental.pallas import tpu as pltpu
from jax.experimental.pallas import tpu_sc as plsc

_INFO = plsc.get_sparse_core_info()
_NC = _INFO.num_cores
_NS = _INFO.num_subcores
_NW = _NC * _NS


@jax.jit
def _gather_t(table, tok_t):
    s, b = tok_t.shape
    d = table.shape[1]
    rows_per_w = b // _NW
    mesh = plsc.VectorSubcoreMesh(core_axis_name="c", subcore_axis_name="s")
    nbuf = 5
    n_groups = s // nbuf

    @functools.partial(
        pl.kernel,
        out_type=jax.ShapeDtypeStruct((s, b, d), jnp.float32),
        mesh=mesh,
        scratch_types=[
            pltpu.VMEM((s, rows_per_w), jnp.int32),
            pltpu.VMEM((nbuf, rows_per_w, d), jnp.float32),
            pltpu.SemaphoreType.DMA((nbuf,)),
            pltpu.SemaphoreType.DMA((nbuf,)),
        ],
    )
    def k(table_hbm, tok_hbm, out_hbm, idx_v, rows_v, gsem, ssem):
        wid = lax.axis_index("s") * _NC + lax.axis_index("c")
        base = wid * rows_per_w
        pltpu.sync_copy(tok_hbm.at[:, pl.ds(base, rows_per_w)], idx_v)

        def gather(j, bf):
            return pltpu.make_async_copy(
                table_hbm.at[idx_v.at[j]], rows_v.at[bf], gsem.at[bf])

        def store(j, bf):
            return pltpu.make_async_copy(
                rows_v.at[bf], out_hbm.at[j, pl.ds(base, rows_per_w)],
                ssem.at[bf])

        for bf in range(nbuf):
            gather(bf, bf).start()

        @pl.loop(0, n_groups)
        def _(g):
            j0 = g * nbuf
            for bf in range(nbuf):
                gather(j0 + bf, bf).wait()
                store(j0 + bf, bf).start()

            @pl.when(g < n_groups - 1)
            def _():
                for bf in range(nbuf):
                    store(j0 + bf, bf).wait()
                    gather(j0 + nbuf + bf, bf).start()

        for bf in range(nbuf):
            store((n_groups - 1) * nbuf + bf, bf).wait()

    return k(table, tok_t)


def kernel(tokens, embeddings):
    out_t = _gather_t(embeddings, tokens.T.astype(jnp.int32))
    return out_t.transpose(1, 0, 2)


# speedup vs baseline: 1.0159x; 1.0159x over previous
import functools

import jax
import jax.numpy as jnp
from jax import lax
from jax.experimental import pallas as pl
from jax.experimental.pallas import tpu as pltpu
from jax.experimental.pallas import tpu_sc as plsc

_INFO = plsc.get_sparse_core_info()
_NC = _INFO.num_cores
_NS = _INFO.num_subcores
_NW = _NC * _NS


@jax.jit
def _gather_t(table, tok_t):
    s, b = tok_t.shape
    d = table.shape[1]
    rows_per_w = b // _NW
    mesh = plsc.VectorSubcoreMesh(core_axis_name="c", subcore_axis_name="s")
    nbuf = 6
    lag = 3

    @functools.partial(
        pl.kernel,
        out_type=jax.ShapeDtypeStruct((s, b, d), jnp.float32),
        mesh=mesh,
        scratch_types=[
            pltpu.VMEM((s, rows_per_w), jnp.int32),
            pltpu.VMEM((nbuf, rows_per_w, d), jnp.float32),
            pltpu.SemaphoreType.DMA((nbuf,)),
            pltpu.SemaphoreType.DMA((nbuf,)),
        ],
    )
    def k(table_hbm, tok_hbm, out_hbm, idx_v, rows_v, gsem, ssem):
        wid = lax.axis_index("s") * _NC + lax.axis_index("c")
        base = wid * rows_per_w
        pltpu.sync_copy(tok_hbm.at[:, pl.ds(base, rows_per_w)], idx_v)

        g_desc, s_desc = {}, {}
        for j in range(s + lag):
            if j < s:
                bf = j % nbuf
                if j >= nbuf:
                    s_desc[j - nbuf].wait()
                g_desc[j] = pltpu.async_copy(
                    table_hbm.at[idx_v.at[j]], rows_v.at[bf], gsem.at[bf])
            i = j - lag
            if i >= 0:
                bi = i % nbuf
                g_desc[i].wait()
                s_desc[i] = pltpu.async_copy(
                    rows_v.at[bi], out_hbm.at[i, pl.ds(base, rows_per_w)],
                    ssem.at[bi])
        for i in range(s - nbuf, s):
            s_desc[i].wait()

    return k(table, tok_t)


def kernel(tokens, embeddings):
    out_t = _gather_t(embeddings, tokens.T.astype(jnp.int32))
    return out_t.transpose(1, 0, 2)


# speedup vs baseline: 1.0183x; 1.0024x over previous
import functools

import jax
import jax.numpy as jnp
from jax import lax
from jax.experimental import pallas as pl
from jax.experimental.pallas import tpu as pltpu
from jax.experimental.pallas import tpu_sc as plsc

_INFO = plsc.get_sparse_core_info()
_NC = _INFO.num_cores
_NS = _INFO.num_subcores
_NW = _NC * _NS


@jax.jit
def _gather_t(table, tok_t):
    s, b = tok_t.shape
    d = table.shape[1]
    rows_per_w = b // _NW
    mesh = plsc.VectorSubcoreMesh(core_axis_name="c", subcore_axis_name="s")
    nbuf = 7
    lag = 3

    @functools.partial(
        pl.kernel,
        out_type=jax.ShapeDtypeStruct((s, b, d), jnp.float32),
        mesh=mesh,
        scratch_types=[
            pltpu.VMEM((s, rows_per_w), jnp.int32),
            pltpu.VMEM((nbuf, rows_per_w, d), jnp.float32),
            pltpu.SemaphoreType.DMA((nbuf,)),
            pltpu.SemaphoreType.DMA((nbuf,)),
        ],
    )
    def k(table_hbm, tok_hbm, out_hbm, idx_v, rows_v, gsem, ssem):
        wid = lax.axis_index("s") * _NC + lax.axis_index("c")
        base = wid * rows_per_w
        pltpu.sync_copy(tok_hbm.at[:, pl.ds(base, rows_per_w)], idx_v)

        g_desc, s_desc = {}, {}
        for j in range(s + lag):
            if j < s:
                bf = j % nbuf
                if j >= nbuf:
                    s_desc[j - nbuf].wait()
                g_desc[j] = pltpu.async_copy(
                    table_hbm.at[idx_v.at[j]], rows_v.at[bf], gsem.at[bf])
            i = j - lag
            if i >= 0:
                bi = i % nbuf
                g_desc[i].wait()
                s_desc[i] = pltpu.async_copy(
                    rows_v.at[bi], out_hbm.at[i, pl.ds(base, rows_per_w)],
                    ssem.at[bi])
        for i in range(s - nbuf, s):
            s_desc[i].wait()

    return k(table, tok_t)


def kernel(tokens, embeddings):
    out_t = _gather_t(embeddings, tokens.T.astype(jnp.int32))
    return out_t.transpose(1, 0, 2)
